# baseline (speedup 1.0000x reference)
import functools

import jax
import jax.numpy as jnp
from jax import lax
from jax.experimental import pallas as pl
from jax.experimental.pallas import tpu as pltpu
from jax.experimental.pallas import tpu_sc as plsc

NC = 2
NS = 16
NW = NC * NS
L = 16
RW = 8


@functools.lru_cache(maxsize=None)
def _face_normals_sc(NP):
    CH = NP // NW
    NSUB = 2
    CHS = CH // NSUB
    mesh = plsc.VectorSubcoreMesh(core_axis_name="c", subcore_axis_name="s")
    out_t = [jax.ShapeDtypeStruct((NP,), jnp.float32)] * 3
    scratch = (
        [pltpu.VMEM((CHS,), jnp.int32)] * 3
        + [pltpu.VMEM((CHS, RW), jnp.float32)] * 3
        + [pltpu.VMEM((CHS,), jnp.float32)] * 3
        + [pltpu.SemaphoreType.DMA]
    )

    @functools.partial(
        pl.kernel, mesh=mesh, out_type=out_t, scratch_types=scratch,
        compiler_params=pltpu.CompilerParams(needs_layout_passes=False,
                                             use_tc_tiling_on_sc=False))
    def k(vtab, f0, f1, f2, onx, ony, onz,
          i0, i1, i2, r0, r1, r2, ox, oy, oz, sem):
        wid = lax.axis_index("s") * NC + lax.axis_index("c")
        base = wid * CH

        lanes = lax.iota(jnp.int32, L)
        c0 = jnp.zeros((L,), jnp.int32)
        c1 = c0 + 1
        c2 = c0 + 2

        def sub(j, carry0):
            sbase = base + j * CHS
            pltpu.sync_copy(f0.at[pl.ds(sbase, CHS)], i0)
            pltpu.sync_copy(f1.at[pl.ds(sbase, CHS)], i1)
            pltpu.sync_copy(f2.at[pl.ds(sbase, CHS)], i2)
            cps = [
                pltpu.async_copy(vtab.at[i0], r0, sem),
                pltpu.async_copy(vtab.at[i1], r1, sem),
                pltpu.async_copy(vtab.at[i2], r2, sem),
            ]
            for c in cps:
                c.wait()

            def step(i, carry):
                s = pl.ds(i * L, L)
                rows = lanes + i * L
                ax0 = plsc.load_gather(r0, [rows, c0])
                ay0 = plsc.load_gather(r0, [rows, c1])
                az0 = plsc.load_gather(r0, [rows, c2])
                ax1 = plsc.load_gather(r1, [rows, c0])
                ay1 = plsc.load_gather(r1, [rows, c1])
                az1 = plsc.load_gather(r1, [rows, c2])
                ax2 = plsc.load_gather(r2, [rows, c0])
                ay2 = plsc.load_gather(r2, [rows, c1])
                az2 = plsc.load_gather(r2, [rows, c2])
                e1x = ax0 - ax1; e1y = ay0 - ay1; e1z = az0 - az1
                e2x = ax2 - ax1; e2y = ay2 - ay1; e2z = az2 - az1
                nx = e2y * e1z - e2z * e1y
                ny = e2z * e1x - e2x * e1z
                nz = e2x * e1y - e2y * e1x
                nn = nx * nx + ny * ny + nz * nz
                ii = jnp.int32(0x5F3759DF) - (plsc.bitcast(nn, jnp.int32) >> 1)
                r = plsc.bitcast(ii, jnp.float32)
                h = nn * jnp.float32(0.5)
                r = r * (jnp.float32(1.5) - (h * r) * r)
                r = r * (jnp.float32(1.5) - (h * r) * r)
                ox[s] = nx * r
                oy[s] = ny * r
                oz[s] = nz * r
                return carry

            lax.fori_loop(0, CHS // L, step, 0, unroll=4)

            pltpu.sync_copy(ox, onx.at[pl.ds(sbase, CHS)])
            pltpu.sync_copy(oy, ony.at[pl.ds(sbase, CHS)])
            pltpu.sync_copy(oz, onz.at[pl.ds(sbase, CHS)])
            return carry0

        lax.fori_loop(0, NSUB, sub, 0)

    return k


def kernel(vertices, faces):
    fi = faces.astype(jnp.int32)
    N = fi.shape[0]
    NP = -(-N // (NW * 128)) * (NW * 128)
    pad = NP - N
    f0 = jnp.pad(fi[:, 0], (0, pad))
    f1 = jnp.pad(fi[:, 1], (0, pad))
    f2 = jnp.pad(fi[:, 2], (0, pad))
    vtab = jnp.pad(vertices, ((0, 0), (0, RW - vertices.shape[1])))
    onx, ony, onz = _face_normals_sc(NP)(vtab, f0, f1, f2)
    return jnp.stack([onx[:N], ony[:N], onz[:N]], axis=-1)


# speedup vs baseline: 2.1059x; 2.1059x over previous
import functools

import jax
import jax.numpy as jnp
from jax import lax
from jax.experimental import pallas as pl
from jax.experimental.pallas import tpu as pltpu
from jax.experimental.pallas import tpu_sc as plsc

NC = 2
NS = 16
NW = NC * NS
L = 16


@functools.lru_cache(maxsize=None)
def _face_normals_sc(NP, VP):
    CH = NP // NW
    SEG = VP // NS
    mesh = plsc.VectorSubcoreMesh(core_axis_name="c", subcore_axis_name="s")
    out_t = [jax.ShapeDtypeStruct((NP,), jnp.float32)] * 3
    scratch = (
        [pltpu.VMEM_SHARED((VP,), jnp.float32)] * 3
        + [pltpu.VMEM((CH,), jnp.int32)] * 3
        + [pltpu.VMEM((CH,), jnp.float32)] * 12
        + [pltpu.SemaphoreType.DMA]
    )

    @functools.partial(
        pl.kernel, mesh=mesh, out_type=out_t, scratch_types=scratch,
        compiler_params=pltpu.CompilerParams(needs_layout_passes=False))
    def k(vx, vy, vz, f0, f1, f2, onx, ony, onz,
          svx, svy, svz,
          i0, i1, i2, x0, y0, z0, x1, y1, z1, x2, y2, z2, ox, oy, oz, sem):
        sid = lax.axis_index("s")
        wid = sid * NC + lax.axis_index("c")
        base = wid * CH

        off = sid * SEG
        seg = pl.ds(off, SEG)
        stg = pl.ds(0, SEG)
        pltpu.sync_copy(vx.at[seg], x0.at[stg])
        pltpu.sync_copy(vy.at[seg], y0.at[stg])
        pltpu.sync_copy(vz.at[seg], z0.at[stg])
        pltpu.sync_copy(x0.at[stg], svx.at[seg])
        pltpu.sync_copy(y0.at[stg], svy.at[seg])
        pltpu.sync_copy(z0.at[stg], svz.at[seg])

        pltpu.sync_copy(f0.at[pl.ds(base, CH)], i0)
        pltpu.sync_copy(f1.at[pl.ds(base, CH)], i1)
        pltpu.sync_copy(f2.at[pl.ds(base, CH)], i2)
        plsc.subcore_barrier()

        cps = [
            pltpu.async_copy(svx.at[i0], x0, sem),
            pltpu.async_copy(svy.at[i0], y0, sem),
            pltpu.async_copy(svz.at[i0], z0, sem),
            pltpu.async_copy(svx.at[i1], x1, sem),
            pltpu.async_copy(svy.at[i1], y1, sem),
            pltpu.async_copy(svz.at[i1], z1, sem),
            pltpu.async_copy(svx.at[i2], x2, sem),
            pltpu.async_copy(svy.at[i2], y2, sem),
            pltpu.async_copy(svz.at[i2], z2, sem),
        ]
        for c in cps:
            c.wait()

        def step(i, carry):
            s = pl.ds(i * L, L)
            ax0 = x0[s]; ay0 = y0[s]; az0 = z0[s]
            ax1 = x1[s]; ay1 = y1[s]; az1 = z1[s]
            ax2 = x2[s]; ay2 = y2[s]; az2 = z2[s]
            e1x = ax0 - ax1; e1y = ay0 - ay1; e1z = az0 - az1
            e2x = ax2 - ax1; e2y = ay2 - ay1; e2z = az2 - az1
            nx = e2y * e1z - e2z * e1y
            ny = e2z * e1x - e2x * e1z
            nz = e2x * e1y - e2y * e1x
            nn = nx * nx + ny * ny + nz * nz
            ii = jnp.int32(0x5F3759DF) - (plsc.bitcast(nn, jnp.int32) >> 1)
            r = plsc.bitcast(ii, jnp.float32)
            h = nn * jnp.float32(0.5)
            r = r * (jnp.float32(1.5) - (h * r) * r)
            r = r * (jnp.float32(1.5) - (h * r) * r)
            ox[s] = nx * r
            oy[s] = ny * r
            oz[s] = nz * r
            return carry

        lax.fori_loop(0, CH // L, step, 0, unroll=4)

        pltpu.sync_copy(ox, onx.at[pl.ds(base, CH)])
        pltpu.sync_copy(oy, ony.at[pl.ds(base, CH)])
        pltpu.sync_copy(oz, onz.at[pl.ds(base, CH)])

    return k


def kernel(vertices, faces):
    fi = faces.astype(jnp.int32)
    N = fi.shape[0]
    V = vertices.shape[0]
    NP = -(-N // (NW * 128)) * (NW * 128)
    VP = -(-V // (NS * 8)) * (NS * 8)
    f0 = jnp.pad(fi[:, 0], (0, NP - N))
    f1 = jnp.pad(fi[:, 1], (0, NP - N))
    f2 = jnp.pad(fi[:, 2], (0, NP - N))
    vx = jnp.pad(vertices[:, 0], (0, VP - V))
    vy = jnp.pad(vertices[:, 1], (0, VP - V))
    vz = jnp.pad(vertices[:, 2], (0, VP - V))
    onx, ony, onz = _face_normals_sc(NP, VP)(vx, vy, vz, f0, f1, f2)
    return jnp.stack([onx[:N], ony[:N], onz[:N]], axis=-1)
